# Initial kernel scaffold; baseline (speedup 1.0000x reference)
import jax
import jax.numpy as jnp
from jax.experimental import pallas as pl


def kernel(x):
    raise NotImplementedError("write your pallas kernel here")


# speedup vs baseline: 135.3198x; 135.3198x over previous
import jax
import jax.numpy as jnp
from jax import lax
from jax.experimental import pallas as pl
from jax.experimental.pallas import tpu as pltpu
from jax.experimental.pallas import tpu_sc as plsc

T = 40
NB = 13
ROWS = T + 8
OUT_LEN = 2097152
LAST_J0 = 4096 - T


def _oadd_body(x_hbm, out_hbm, buf, acc):
    wid = lax.axis_index("c") * 16 + lax.axis_index("s")
    b = wid >> 3
    s = wid & 7

    def block(i, carry):
        t = s * NB + i
        j0 = jnp.minimum(3 + T * t, LAST_J0)
        a0 = pl.multiple_of((j0 - 3) & ~7, 8)
        d = (j0 - 3) & 7
        pltpu.sync_copy(x_hbm.at[b, pl.ds(a0, ROWS), :], buf)

        def add_chunk(g, c2):
            i2 = g >> 5
            c = (g & 31) * 16
            r = d + i2
            v = (
                buf[r + 3, pl.ds(c, 16)]
                + buf[r + 2, pl.ds(512 + c, 16)]
                + buf[r + 1, pl.ds(1024 + c, 16)]
                + buf[r, pl.ds(1536 + c, 16)]
            )
            acc[pl.ds(g * 16, 16)] = v * 0.25
            return c2

        lax.fori_loop(0, T * 32, add_chunk, 0)
        pltpu.sync_copy(acc, out_hbm.at[b, 0, pl.ds(512 * j0 - 768, 512 * T)])
        return carry

    lax.fori_loop(0, NB, block, 0)

    @pl.when(s == 0)
    def _start_edge():
        pltpu.sync_copy(x_hbm.at[b, pl.ds(0, 8), :], buf.at[pl.ds(0, 8), :])

        def e1(g, c2):
            c = g * 16
            acc[pl.ds(c, 16)] = (
                buf[1, pl.ds(256 + c, 16)] + buf[0, pl.ds(768 + c, 16)]
            ) * 0.5
            return c2

        lax.fori_loop(0, 16, e1, 0)

        def e2(g, c2):
            c = g * 16
            acc[pl.ds(256 + c, 16)] = (
                buf[2, pl.ds(c, 16)]
                + buf[1, pl.ds(512 + c, 16)]
                + buf[0, pl.ds(1024 + c, 16)]
            ) / 3.0
            return c2

        lax.fori_loop(0, 32, e2, 0)
        pltpu.sync_copy(acc.at[pl.ds(0, 768)], out_hbm.at[b, 0, pl.ds(0, 768)])

    @pl.when(s == 7)
    def _end_edge():
        pltpu.sync_copy(x_hbm.at[b, pl.ds(4088, 8), :], buf.at[pl.ds(0, 8), :])

        def e3(g, c2):
            c = g * 16
            acc[pl.ds(c, 16)] = (
                buf[7, pl.ds(512 + c, 16)]
                + buf[6, pl.ds(1024 + c, 16)]
                + buf[5, pl.ds(1536 + c, 16)]
            ) / 3.0
            return c2

        lax.fori_loop(0, 32, e3, 0)

        def e4(g, c2):
            c = g * 16
            acc[pl.ds(512 + c, 16)] = (
                buf[7, pl.ds(1024 + c, 16)] + buf[6, pl.ds(1536 + c, 16)]
            ) * 0.5
            return c2

        lax.fori_loop(0, 16, e4, 0)
        pltpu.sync_copy(
            acc.at[pl.ds(0, 768)], out_hbm.at[b, 0, pl.ds(OUT_LEN - 768, 768)]
        )


@jax.jit
def kernel(x):
    mesh = plsc.VectorSubcoreMesh(core_axis_name="c", subcore_axis_name="s")
    run = pl.kernel(
        _oadd_body,
        out_type=jax.ShapeDtypeStruct((4, 1, OUT_LEN), jnp.float32),
        mesh=mesh,
        scratch_types=[
            pltpu.VMEM((ROWS, 2048), jnp.float32),
            pltpu.VMEM((T * 512,), jnp.float32),
        ],
    )
    return run(x).reshape(4, OUT_LEN)
